# baseline (speedup 1.0000x reference)
import functools

import jax
import jax.numpy as jnp
from jax import lax
from jax.experimental import pallas as pl
from jax.experimental.pallas import tpu as pltpu
from jax.experimental.pallas import tpu_sc as plsc

N = 10000
E = 320000
D = 128

NC, NS = 2, 16
NW = NC * NS
LANES = 16


_BN = 2000


def _proj_body(x_ref, w1_ref, b1_ref, p_ref, q_ref):
    w1a = w1_ref[:D, :]
    w1b = w1_ref[D:, :]
    x = x_ref[...]
    p_ref[...] = (
        jnp.dot(x, w1a - w1b, preferred_element_type=jnp.float32) + b1_ref[...]
    )
    q_ref[...] = jnp.dot(x, w1b, preferred_element_type=jnp.float32)


def _project(x, w1, b1):
    grid = (N // _BN,)
    return pl.pallas_call(
        _proj_body,
        grid=grid,
        in_specs=[
            pl.BlockSpec((_BN, D), lambda i: (i, 0)),
            pl.BlockSpec((2 * D, D), lambda i: (0, 0)),
            pl.BlockSpec((1, D), lambda i: (0, 0)),
        ],
        out_specs=[
            pl.BlockSpec((_BN, D), lambda i: (i, 0)),
            pl.BlockSpec((_BN, D), lambda i: (i, 0)),
        ],
        out_shape=[
            jax.ShapeDtypeStruct((N, D), jnp.float32),
            jax.ShapeDtypeStruct((N, D), jnp.float32),
        ],
    )(x, w1, b1.reshape(1, D))


_GCH = 200
_EPW = E // NW
_NGCH = _EPW // _GCH


def _gather_body(p_hbm, q_hbm, dst_hbm, src_hbm, pd_hbm, qs_hbm,
                 didx, sidx, pbuf, qbuf, sem_p, sem_q):
    wid = lax.axis_index("s") * NC + lax.axis_index("c")
    ebase = wid * _EPW

    def chunk(i, _):
        base = ebase + i * _GCH
        pltpu.sync_copy(dst_hbm.at[pl.ds(base, _GCH)], didx)
        pltpu.sync_copy(src_hbm.at[pl.ds(base, _GCH)], sidx)
        cp = pltpu.async_copy(p_hbm.at[didx], pbuf, sem_p)
        cq = pltpu.async_copy(q_hbm.at[sidx], qbuf, sem_q)
        cp.wait()
        cq.wait()
        pltpu.sync_copy(pbuf, pd_hbm.at[pl.ds(base, _GCH)])
        pltpu.sync_copy(qbuf, qs_hbm.at[pl.ds(base, _GCH)])
        return _

    lax.fori_loop(0, _NGCH, chunk, 0)


_gather = functools.partial(
    pl.kernel,
    mesh=plsc.VectorSubcoreMesh(
        core_axis_name="c", subcore_axis_name="s", num_cores=NC, num_subcores=NS
    ),
    out_type=[
        jax.ShapeDtypeStruct((E, D), jnp.float32),
        jax.ShapeDtypeStruct((E, D), jnp.float32),
    ],
    scratch_types=[
        pltpu.VMEM((_GCH,), jnp.int32),
        pltpu.VMEM((_GCH,), jnp.int32),
        pltpu.VMEM((_GCH, D), jnp.float32),
        pltpu.VMEM((_GCH, D), jnp.float32),
        pltpu.SemaphoreType.DMA,
        pltpu.SemaphoreType.DMA,
    ],
    compiler_params=pltpu.CompilerParams(needs_layout_passes=False),
)(_gather_body)


_BE = 3200
_CG = D // LANES


def _mlp_body(pd_ref, qs_ref, w2_ref, b2_ref, z_ref):
    h = jnp.maximum(pd_ref[...] + qs_ref[...], 0.0)
    z = jnp.dot(h, w2_ref[...], preferred_element_type=jnp.float32) + b2_ref[...]
    zr = z.reshape(_BE // 8, 8, D)
    lane = lax.broadcasted_iota(jnp.int32, (_BE // 8, D), 1)
    for g in range(_CG):
        accv = jnp.zeros((_BE // 8, D), jnp.float32)
        for j in range(8):
            row = zr[:, j, :]
            shifted = pltpu.roll(row, ((j - g) % 8) * LANES, axis=1)
            m = (lane >= j * LANES) & (lane < (j + 1) * LANES)
            accv = jnp.where(m, shifted, accv)
        z_ref[g, :, :] = accv


def _edge_mlp(pd, qs, w2, b2):
    grid = (E // _BE,)
    return pl.pallas_call(
        _mlp_body,
        grid=grid,
        in_specs=[
            pl.BlockSpec((_BE, D), lambda i: (i, 0)),
            pl.BlockSpec((_BE, D), lambda i: (i, 0)),
            pl.BlockSpec((D, D), lambda i: (0, 0)),
            pl.BlockSpec((1, D), lambda i: (0, 0)),
        ],
        out_specs=pl.BlockSpec((_CG, _BE // 8, D), lambda i: (0, i, 0)),
        out_shape=jax.ShapeDtypeStruct((_CG, E // 8, D), jnp.float32),
    )(pd, qs, w2, b2.reshape(1, D))


_TS = 4
_ESL = E // _TS
_SCH = 800
_SCHR = _SCH // 8
_NSCH = _ESL // _SCH
_NH = N // 2
_NPAD = 10240
_DUMMY = _NH

_NEG = float("-inf")


def _scatter_body(z_hbm, dst_hbm, part_hbm,
                  dbufA, dbufB, zbufA, zbufB, acc, tmpb, semd, semz):
    c = lax.axis_index("c")
    sc = lax.axis_index("s")
    g = sc & 7
    t = (sc >> 3) + 2 * c
    ebase = t * _ESL
    zw0 = (g * E + ebase) * LANES
    pw0 = (t * _CG + g) * (_NPAD * LANES)
    dbufs = (dbufA, dbufB)
    zbufs = (zbufA, zbufB)

    for h in range(2):
        nbase = h * _NH

        def init_slot(r, _):
            acc[pl.ds(r * LANES, LANES)] = jnp.full((LANES,), _NEG, jnp.float32)
            return _
        lax.fori_loop(0, _NH + 1, init_slot, 0)

        pltpu.async_copy(dst_hbm.at[pl.ds(ebase, _SCH)], dbufA, semd)
        pltpu.async_copy(z_hbm.at[pl.ds(zw0, _SCH * LANES)], zbufA, semz)

        def make_half(half):
            dbuf = dbufs[half]
            zbuf = zbufs[half]

            def run(ci):
                pltpu.make_async_copy(
                    dst_hbm.at[pl.ds(ebase, _SCH)], dbuf, semd).wait()
                pltpu.make_async_copy(
                    z_hbm.at[pl.ds(zw0, _SCH * LANES)], zbuf, semz).wait()
                nxt = ci + 1

                @pl.when(nxt < _NSCH)
                def _():
                    pltpu.async_copy(
                        dst_hbm.at[pl.ds(ebase + nxt * _SCH, _SCH)],
                        dbufs[1 - half], semd)
                    pltpu.async_copy(
                        z_hbm.at[pl.ds(zw0 + nxt * _SCH * LANES, _SCH * LANES)],
                        zbufs[1 - half], semz)

                def group(jg, carry):
                    d = dbuf[pl.ds(jg * LANES, LANES)]
                    r = d - nbase
                    valid = (r >= 0) & (r < _NH)
                    lidx = lax.iota(jnp.int32, LANES)
                    rsel = jnp.where(valid, r, _DUMMY + lidx)
                    rs16 = rsel * LANES
                    zo = jg * (LANES * LANES) + lidx * LANES
                    plsc.store_scatter(tmpb, [rsel], lidx)
                    rb = plsc.load_gather(tmpb, [rsel])
                    nd = plsc.all_reduce_population_count(rb != lidx)[0]
                    for cc in range(LANES):
                        av = plsc.load_gather(acc, [rs16 + cc])
                        zv = plsc.load_gather(zbuf, [zo + cc])
                        plsc.store_scatter(acc, [rs16 + cc], jnp.maximum(av, zv))

                    @pl.when(nd > 0)
                    def _():
                        for l in range(LANES):
                            rk = rs16[l]
                            zv2 = zbuf[pl.ds(jg * (LANES * LANES) + l * LANES,
                                             LANES)]
                            av2 = acc[pl.ds(rk, LANES)]
                            acc[pl.ds(rk, LANES)] = jnp.maximum(av2, zv2)
                    return carry
                lax.fori_loop(0, _SCH // LANES, group, 0)
            return run

        def chunk_pair(ip, _):
            make_half(0)(ip * 2)
            make_half(1)(ip * 2 + 1)
            return _
        lax.fori_loop(0, _NSCH // 2, chunk_pair, 0)

        pltpu.sync_copy(acc.at[pl.ds(0, _NH * LANES)],
                        part_hbm.at[pl.ds(pw0 + nbase * LANES, _NH * LANES)])


_scatter = functools.partial(
    pl.kernel,
    mesh=plsc.VectorSubcoreMesh(
        core_axis_name="c", subcore_axis_name="s", num_cores=NC, num_subcores=NS
    ),
    out_type=jax.ShapeDtypeStruct((_TS * _CG * _NPAD * LANES,), jnp.float32),
    scratch_types=[
        pltpu.VMEM((_SCH,), jnp.int32),
        pltpu.VMEM((_SCH,), jnp.int32),
        pltpu.VMEM((_SCH * LANES,), jnp.float32),
        pltpu.VMEM((_SCH * LANES,), jnp.float32),
        pltpu.VMEM(((_NH + LANES) * LANES,), jnp.float32),
        pltpu.VMEM((_NH + LANES,), jnp.int32),
        pltpu.SemaphoreType.DMA,
        pltpu.SemaphoreType.DMA,
    ],
    compiler_params=pltpu.CompilerParams(needs_layout_passes=False),
)(_scatter_body)


_BNM = 2048
_BPR = _BNM // 8


def _merge_body(*refs):
    out_ref = refs[-1]
    for g in range(_CG):
        m = refs[g * _TS][...]
        for t in range(1, _TS):
            m = jnp.maximum(m, refs[g * _TS + t][...])
        out_ref[g, :, :] = jnp.where(m == _NEG, jnp.float32(0.0), m)


def _merge(part2d):
    grid = (_NPAD // _BNM,)
    specs = []
    for g in range(_CG):
        for t in range(_TS):
            base_blocks = (t * _CG + g) * (_NPAD // _BNM)
            specs.append(
                pl.BlockSpec(
                    (_BPR, 128),
                    functools.partial(
                        lambda bb, i: (bb + i, 0), base_blocks),
                )
            )
    return pl.pallas_call(
        _merge_body,
        grid=grid,
        in_specs=specs,
        out_specs=pl.BlockSpec((_CG, _BPR, 128), lambda i: (0, i, 0)),
        out_shape=jax.ShapeDtypeStruct((_CG, _NPAD // 8, 128), jnp.float32),
    )(*([part2d] * (_CG * _TS)))


@jax.jit
def kernel(x, edge_index, W1, b1, W2, b2):
    ei = edge_index.astype(jnp.int32)
    src = ei[0]
    dst = ei[1]
    p, q = _project(x, W1, b1)
    pd, qs = _gather(p, q, dst, src)
    z = _edge_mlp(pd, qs, W2, b2)
    part = _scatter(z.reshape(-1), dst)
    op = _merge(part.reshape(_TS * _CG * _NPAD // 8, 128))
    out = jnp.transpose(
        op.reshape(_CG, _NPAD // 8, 8, LANES), (1, 2, 0, 3)
    ).reshape(_NPAD, D)
    return out[:N]


# speedup vs baseline: 2.0714x; 2.0714x over previous
import functools

import jax
import jax.numpy as jnp
from jax import lax
from jax.experimental import pallas as pl
from jax.experimental.pallas import tpu as pltpu
from jax.experimental.pallas import tpu_sc as plsc

N = 10000
E = 320000
D = 128

NC, NS = 2, 16
NW = NC * NS
LANES = 16


_BN = 2000


def _proj_body(x_ref, w1_ref, b1_ref, p_ref, q_ref):
    w1a = w1_ref[:D, :]
    w1b = w1_ref[D:, :]
    x = x_ref[...]
    p_ref[...] = (
        jnp.dot(x, w1a - w1b, preferred_element_type=jnp.float32) + b1_ref[...]
    )
    q_ref[...] = jnp.dot(x, w1b, preferred_element_type=jnp.float32)


def _project(x, w1, b1):
    grid = (N // _BN,)
    return pl.pallas_call(
        _proj_body,
        grid=grid,
        in_specs=[
            pl.BlockSpec((_BN, D), lambda i: (i, 0)),
            pl.BlockSpec((2 * D, D), lambda i: (0, 0)),
            pl.BlockSpec((1, D), lambda i: (0, 0)),
        ],
        out_specs=[
            pl.BlockSpec((_BN, D), lambda i: (i, 0)),
            pl.BlockSpec((_BN, D), lambda i: (i, 0)),
        ],
        out_shape=[
            jax.ShapeDtypeStruct((N, D), jnp.float32),
            jax.ShapeDtypeStruct((N, D), jnp.float32),
        ],
    )(x, w1, b1.reshape(1, D))


_GCH = 200
_EPW = E // NW
_NGCH = _EPW // _GCH


def _gather_body(p_hbm, q_hbm, dst_hbm, src_hbm, pd_hbm, qs_hbm,
                 didx0, sidx0, pbuf0, qbuf0, didx1, sidx1, pbuf1, qbuf1,
                 semi0, semi1, semg0, semg1, semw0, semw1):
    wid = lax.axis_index("s") * NC + lax.axis_index("c")
    ebase = wid * _EPW
    sets = ((didx0, sidx0, pbuf0, qbuf0, semi0, semg0, semw0),
            (didx1, sidx1, pbuf1, qbuf1, semi1, semg1, semw1))

    pltpu.async_copy(dst_hbm.at[pl.ds(ebase, _GCH)], didx0, semi0)
    pltpu.async_copy(src_hbm.at[pl.ds(ebase, _GCH)], sidx0, semi0)

    def make_run(half):
        didx, sidx, pbuf, qbuf, semi, semg, semw = sets[half]

        def run(ci):
            base = ebase + ci * _GCH
            pltpu.make_async_copy(
                dst_hbm.at[pl.ds(base, _GCH)], didx, semi).wait()
            pltpu.make_async_copy(
                src_hbm.at[pl.ds(base, _GCH)], sidx, semi).wait()

            @pl.when(ci >= 2)
            def _():
                pltpu.make_async_copy(
                    pbuf, pd_hbm.at[pl.ds(base, _GCH)], semw).wait()
                pltpu.make_async_copy(
                    qbuf, qs_hbm.at[pl.ds(base, _GCH)], semw).wait()

            cp = pltpu.async_copy(p_hbm.at[didx], pbuf, semg)
            cq = pltpu.async_copy(q_hbm.at[sidx], qbuf, semg)

            nxt = ci + 1

            @pl.when(nxt < _NGCH)
            def _():
                off = ebase + nxt * _GCH
                o = sets[1 - half]
                pltpu.async_copy(dst_hbm.at[pl.ds(off, _GCH)], o[0], o[4])
                pltpu.async_copy(src_hbm.at[pl.ds(off, _GCH)], o[1], o[4])

            cp.wait()
            cq.wait()
            pltpu.async_copy(pbuf, pd_hbm.at[pl.ds(base, _GCH)], semw)
            pltpu.async_copy(qbuf, qs_hbm.at[pl.ds(base, _GCH)], semw)
        return run

    def pair(ip, _):
        make_run(0)(ip * 2)
        make_run(1)(ip * 2 + 1)
        return _
    lax.fori_loop(0, _NGCH // 2, pair, 0)

    for half in range(2):
        didx, sidx, pbuf, qbuf, semi, semg, semw = sets[half]
        pltpu.make_async_copy(pbuf, pd_hbm.at[pl.ds(ebase, _GCH)], semw).wait()
        pltpu.make_async_copy(qbuf, qs_hbm.at[pl.ds(ebase, _GCH)], semw).wait()


_gather = functools.partial(
    pl.kernel,
    mesh=plsc.VectorSubcoreMesh(
        core_axis_name="c", subcore_axis_name="s", num_cores=NC, num_subcores=NS
    ),
    out_type=[
        jax.ShapeDtypeStruct((E, D), jnp.float32),
        jax.ShapeDtypeStruct((E, D), jnp.float32),
    ],
    scratch_types=[
        pltpu.VMEM((_GCH,), jnp.int32),
        pltpu.VMEM((_GCH,), jnp.int32),
        pltpu.VMEM((_GCH, D), jnp.float32),
        pltpu.VMEM((_GCH, D), jnp.float32),
        pltpu.VMEM((_GCH,), jnp.int32),
        pltpu.VMEM((_GCH,), jnp.int32),
        pltpu.VMEM((_GCH, D), jnp.float32),
        pltpu.VMEM((_GCH, D), jnp.float32),
        pltpu.SemaphoreType.DMA,
        pltpu.SemaphoreType.DMA,
        pltpu.SemaphoreType.DMA,
        pltpu.SemaphoreType.DMA,
        pltpu.SemaphoreType.DMA,
        pltpu.SemaphoreType.DMA,
    ],
    compiler_params=pltpu.CompilerParams(needs_layout_passes=False),
)(_gather_body)


_BE = 3200
_CG = D // LANES


def _mlp_body(pd_ref, qs_ref, w2_ref, b2_ref, z_ref):
    h = jnp.maximum(pd_ref[...] + qs_ref[...], 0.0)
    z = jnp.dot(h, w2_ref[...], preferred_element_type=jnp.float32) + b2_ref[...]
    zr = z.reshape(_BE // 8, 8, D)
    lane = lax.broadcasted_iota(jnp.int32, (_BE // 8, D), 1)
    for g in range(_CG):
        accv = jnp.zeros((_BE // 8, D), jnp.float32)
        for j in range(8):
            row = zr[:, j, :]
            shifted = pltpu.roll(row, ((j - g) % 8) * LANES, axis=1)
            m = (lane >= j * LANES) & (lane < (j + 1) * LANES)
            accv = jnp.where(m, shifted, accv)
        z_ref[g, :, :] = accv


def _edge_mlp(pd, qs, w2, b2):
    grid = (E // _BE,)
    return pl.pallas_call(
        _mlp_body,
        grid=grid,
        in_specs=[
            pl.BlockSpec((_BE, D), lambda i: (i, 0)),
            pl.BlockSpec((_BE, D), lambda i: (i, 0)),
            pl.BlockSpec((D, D), lambda i: (0, 0)),
            pl.BlockSpec((1, D), lambda i: (0, 0)),
        ],
        out_specs=pl.BlockSpec((_CG, _BE // 8, D), lambda i: (0, i, 0)),
        out_shape=jax.ShapeDtypeStruct((_CG, E // 8, D), jnp.float32),
    )(pd, qs, w2, b2.reshape(1, D))


_TS = 4
_ESL = E // _TS
_SCH = 800
_SCHR = _SCH // 8
_NSCH = _ESL // _SCH
_NH = N // 2
_NPAD = 10240
_DUMMY = _NH

_NEG = float("-inf")


def _scatter_body(z_hbm, dst_hbm, part_hbm,
                  dbufA, dbufB, zbufA, zbufB, acc, semd, semz):
    c = lax.axis_index("c")
    sc = lax.axis_index("s")
    g = sc & 7
    t = (sc >> 3) + 2 * c
    ebase = t * _ESL
    zw0 = (g * E + ebase) * LANES
    pw0 = (t * _CG + g) * (_NPAD * LANES)
    dbufs = (dbufA, dbufB)
    zbufs = (zbufA, zbufB)

    for h in range(2):
        nbase = h * _NH

        def init_slot(r, _):
            acc[pl.ds(r * LANES, LANES)] = jnp.full((LANES,), _NEG, jnp.float32)
            return _
        lax.fori_loop(0, _NH + 1, init_slot, 0)

        pltpu.async_copy(dst_hbm.at[pl.ds(ebase, _SCH)], dbufA, semd)
        pltpu.async_copy(z_hbm.at[pl.ds(zw0, _SCH * LANES)], zbufA, semz)

        def make_half(half):
            dbuf = dbufs[half]
            zbuf = zbufs[half]

            def run(ci):
                pltpu.make_async_copy(
                    dst_hbm.at[pl.ds(ebase, _SCH)], dbuf, semd).wait()
                pltpu.make_async_copy(
                    z_hbm.at[pl.ds(zw0, _SCH * LANES)], zbuf, semz).wait()
                nxt = ci + 1

                @pl.when(nxt < _NSCH)
                def _():
                    pltpu.async_copy(
                        dst_hbm.at[pl.ds(ebase + nxt * _SCH, _SCH)],
                        dbufs[1 - half], semd)
                    pltpu.async_copy(
                        z_hbm.at[pl.ds(zw0 + nxt * _SCH * LANES, _SCH * LANES)],
                        zbufs[1 - half], semz)

                def group(jg, _):
                    d = dbuf[pl.ds(jg * LANES, LANES)]
                    r = d - nbase
                    valid = (r >= 0) & (r < _NH)
                    rsel = jnp.where(valid, r, _DUMMY) * LANES
                    for l in range(LANES):
                        rk = rsel[l]
                        zv = zbuf[pl.ds(jg * (LANES * LANES) + l * LANES, LANES)]
                        av = acc[pl.ds(rk, LANES)]
                        acc[pl.ds(rk, LANES)] = jnp.maximum(av, zv)
                    return _
                lax.fori_loop(0, _SCH // LANES, group, 0, unroll=2)
            return run

        def chunk_pair(ip, _):
            make_half(0)(ip * 2)
            make_half(1)(ip * 2 + 1)
            return _
        lax.fori_loop(0, _NSCH // 2, chunk_pair, 0)

        pltpu.sync_copy(acc.at[pl.ds(0, _NH * LANES)],
                        part_hbm.at[pl.ds(pw0 + nbase * LANES, _NH * LANES)])


_scatter = functools.partial(
    pl.kernel,
    mesh=plsc.VectorSubcoreMesh(
        core_axis_name="c", subcore_axis_name="s", num_cores=NC, num_subcores=NS
    ),
    out_type=jax.ShapeDtypeStruct((_TS * _CG * _NPAD * LANES,), jnp.float32),
    scratch_types=[
        pltpu.VMEM((_SCH,), jnp.int32),
        pltpu.VMEM((_SCH,), jnp.int32),
        pltpu.VMEM((_SCH * LANES,), jnp.float32),
        pltpu.VMEM((_SCH * LANES,), jnp.float32),
        pltpu.VMEM(((_NH + 1) * LANES,), jnp.float32),
        pltpu.SemaphoreType.DMA,
        pltpu.SemaphoreType.DMA,
    ],
    compiler_params=pltpu.CompilerParams(needs_layout_passes=False),
)(_scatter_body)


_BNM = 2048
_BPR = _BNM // 8


def _merge_body(*refs):
    out_ref = refs[-1]
    for g in range(_CG):
        m = refs[g * _TS][...]
        for t in range(1, _TS):
            m = jnp.maximum(m, refs[g * _TS + t][...])
        out_ref[g, :, :] = jnp.where(m == _NEG, jnp.float32(0.0), m)


def _merge(part2d):
    grid = (_NPAD // _BNM,)
    specs = []
    for g in range(_CG):
        for t in range(_TS):
            base_blocks = (t * _CG + g) * (_NPAD // _BNM)
            specs.append(
                pl.BlockSpec(
                    (_BPR, 128),
                    functools.partial(
                        lambda bb, i: (bb + i, 0), base_blocks),
                )
            )
    return pl.pallas_call(
        _merge_body,
        grid=grid,
        in_specs=specs,
        out_specs=pl.BlockSpec((_CG, _BPR, 128), lambda i: (0, i, 0)),
        out_shape=jax.ShapeDtypeStruct((_CG, _NPAD // 8, 128), jnp.float32),
    )(*([part2d] * (_CG * _TS)))


@jax.jit
def kernel(x, edge_index, W1, b1, W2, b2):
    ei = edge_index.astype(jnp.int32)
    src = ei[0]
    dst = ei[1]
    p, q = _project(x, W1, b1)
    pd, qs = _gather(p, q, dst, src)
    z = _edge_mlp(pd, qs, W2, b2)
    part = _scatter(z.reshape(-1), dst)
    op = _merge(part.reshape(_TS * _CG * _NPAD // 8, 128))
    out = jnp.transpose(
        op.reshape(_CG, _NPAD // 8, 8, LANES), (1, 2, 0, 3)
    ).reshape(_NPAD, D)
    return out[:N]
